# baseline (speedup 1.0000x reference)
import jax
import jax.numpy as jnp
from jax import lax
from jax.experimental import pallas as pl
from jax.experimental.pallas import tpu as pltpu

V = 100000
D = 128
C = 20
N = 128
VB = 4096
NBLK = (V + VB - 1) // VB
PADV = NBLK * VB


def _h_kernel(idx_ref, emb_ref, w1_ref, b1_ref, h_ref, g_ref, sem):
    for p in range(C):
        pltpu.make_async_copy(
            emb_ref.at[pl.ds(idx_ref[p], 1), :],
            g_ref.at[pl.ds(p, 1), :],
            sem,
        ).start()
    for p in range(C):
        pltpu.make_async_copy(
            emb_ref.at[pl.ds(idx_ref[p], 1), :],
            g_ref.at[pl.ds(p, 1), :],
            sem,
        ).wait()
    acc = b1_ref[...].astype(jnp.float32)
    for p in range(C):
        acc = acc + lax.dot_general(
            g_ref[pl.ds(p, 1), :],
            w1_ref[:, pl.ds(p * D, D)],
            (((1,), (1,)), ((), ())),
            preferred_element_type=jnp.float32,
        )
    h_ref[...] = jnp.maximum(acc, 0.0)


def _logits_kernel(h_ref, w2_ref, b2_ref, lg_ref, m_ref, s_ref):
    i = pl.program_id(0)
    lb = lax.dot_general(
        h_ref[...],
        w2_ref[...],
        (((1,), (1,)), ((), ())),
        preferred_element_type=jnp.float32,
    ) + b2_ref[...]
    cols = i * VB + lax.broadcasted_iota(jnp.int32, (1, VB), 1)
    lb = jnp.where(cols < V, lb, -1e30)
    lg_ref[...] = lb
    bm = jnp.max(lb, axis=1, keepdims=True)
    bs = jnp.sum(jnp.exp(lb - bm), axis=1, keepdims=True)
    m_ref[...] = jnp.broadcast_to(bm, (1, 1, N))
    s_ref[...] = jnp.broadcast_to(bs, (1, 1, N))


def _norm_kernel(m_ref, s_ref, lg_ref, out_ref):
    ms = m_ref[:, 0, :]
    ss = s_ref[:, 0, :]
    mvec = jnp.max(ms, axis=0, keepdims=True)
    svec = jnp.sum(ss * jnp.exp(ms - mvec), axis=0, keepdims=True)
    logz = mvec + jnp.log(svec)
    out_ref[...] = lg_ref[...] - logz[0:1, 0:1]


def kernel(inputs, emb, W1, b1, W2, b2):
    b1r = b1.reshape(1, N)
    b2r = b2.reshape(1, V)

    h = pl.pallas_call(
        _h_kernel,
        in_specs=[
            pl.BlockSpec(memory_space=pltpu.MemorySpace.SMEM),
            pl.BlockSpec(memory_space=pltpu.MemorySpace.HBM),
            pl.BlockSpec((N, C * D), lambda: (0, 0)),
            pl.BlockSpec((1, N), lambda: (0, 0)),
        ],
        out_specs=pl.BlockSpec((1, N), lambda: (0, 0)),
        out_shape=jax.ShapeDtypeStruct((1, N), jnp.float32),
        scratch_shapes=[
            pltpu.VMEM((C, D), jnp.float32),
            pltpu.SemaphoreType.DMA,
        ],
    )(inputs, emb, W1, b1r)

    lg, m, s = pl.pallas_call(
        _logits_kernel,
        grid=(NBLK,),
        in_specs=[
            pl.BlockSpec((1, N), lambda i: (0, 0)),
            pl.BlockSpec((VB, D), lambda i: (i, 0)),
            pl.BlockSpec((1, VB), lambda i: (0, i)),
        ],
        out_specs=[
            pl.BlockSpec((1, VB), lambda i: (0, i)),
            pl.BlockSpec((1, 1, N), lambda i: (i, 0, 0)),
            pl.BlockSpec((1, 1, N), lambda i: (i, 0, 0)),
        ],
        out_shape=[
            jax.ShapeDtypeStruct((1, PADV), jnp.float32),
            jax.ShapeDtypeStruct((NBLK, 1, N), jnp.float32),
            jax.ShapeDtypeStruct((NBLK, 1, N), jnp.float32),
        ],
        compiler_params=pltpu.CompilerParams(
            dimension_semantics=("parallel",),
        ),
    )(h, W2, b2r)

    out = pl.pallas_call(
        _norm_kernel,
        grid=(NBLK,),
        in_specs=[
            pl.BlockSpec((NBLK, 1, N), lambda i: (0, 0, 0)),
            pl.BlockSpec((NBLK, 1, N), lambda i: (0, 0, 0)),
            pl.BlockSpec((1, VB), lambda i: (0, i)),
        ],
        out_specs=pl.BlockSpec((1, VB), lambda i: (0, i)),
        out_shape=jax.ShapeDtypeStruct((1, PADV), jnp.float32),
        compiler_params=pltpu.CompilerParams(
            dimension_semantics=("parallel",),
        ),
    )(m, s, lg)

    return out[:, :V]


# speedup vs baseline: 1.2923x; 1.2923x over previous
import jax
import jax.numpy as jnp
from jax import lax
from jax.experimental import pallas as pl
from jax.experimental.pallas import tpu as pltpu

V = 100000
D = 128
C = 20
N = 128
VB = 3200
NI = 8
NJ = 4
PADV = NI * NJ * VB


def _fused_kernel(idx_ref, emb_ref, w1_ref, b1_ref, w2_ref, b2_ref,
                  out_ref, g_ref, h_ref, sem):
    i = pl.program_id(0)
    j = pl.program_id(1)

    @pl.when((i == 0) & (j == 0))
    def _gather_and_hidden():
        for p in range(C):
            pltpu.make_async_copy(
                emb_ref.at[pl.ds(idx_ref[p], 1), :],
                g_ref.at[pl.ds(p, 1), :],
                sem,
            ).start()
        for p in range(C):
            pltpu.make_async_copy(
                emb_ref.at[pl.ds(idx_ref[p], 1), :],
                g_ref.at[pl.ds(p, 1), :],
                sem,
            ).wait()
        acc = b1_ref[...].astype(jnp.float32)
        for p in range(C):
            acc = acc + lax.dot_general(
                g_ref[pl.ds(p, 1), :],
                w1_ref[:, pl.ds(p * D, D)],
                (((1,), (1,)), ((), ())),
                preferred_element_type=jnp.float32,
            )
        h_ref[...] = jnp.maximum(acc, 0.0)

    lb = lax.dot_general(
        h_ref[...],
        w2_ref[...],
        (((1,), (1,)), ((), ())),
        preferred_element_type=jnp.float32,
    ) + b2_ref[...]

    last = (i == NI - 1) & (j == NJ - 1)

    @pl.when(jnp.logical_not(last))
    def _store():
        out_ref[pl.ds(i, 1), pl.ds(j * VB, VB)] = lb

    @pl.when(last)
    def _store_masked_and_normalize():
        cols = (NI * NJ - 1) * VB + lax.broadcasted_iota(jnp.int32, (1, VB), 1)
        out_ref[pl.ds(i, 1), pl.ds(j * VB, VB)] = jnp.where(cols < V, lb, -1e30)
        scr = out_ref[...]
        m = jnp.max(jnp.max(scr, axis=1, keepdims=True), axis=0, keepdims=True)
        e = jnp.exp(scr - m)
        s = jnp.sum(jnp.sum(e, axis=1, keepdims=True), axis=0, keepdims=True)
        out_ref[...] = scr - (m + jnp.log(s))


def kernel(inputs, emb, W1, b1, W2, b2):
    b1r = b1.reshape(1, N)
    b2r = b2.reshape(1, V)
    out = pl.pallas_call(
        _fused_kernel,
        grid=(NI, NJ),
        in_specs=[
            pl.BlockSpec(memory_space=pltpu.MemorySpace.SMEM),
            pl.BlockSpec(memory_space=pltpu.MemorySpace.HBM),
            pl.BlockSpec((N, C * D), lambda i, j: (0, 0)),
            pl.BlockSpec((1, N), lambda i, j: (0, 0)),
            pl.BlockSpec((VB, D), lambda i, j: (i * NJ + j, 0)),
            pl.BlockSpec((1, VB), lambda i, j: (0, i * NJ + j)),
        ],
        out_specs=pl.BlockSpec((NI, NJ * VB), lambda i, j: (0, 0)),
        out_shape=jax.ShapeDtypeStruct((NI, NJ * VB), jnp.float32),
        scratch_shapes=[
            pltpu.VMEM((C, D), jnp.float32),
            pltpu.VMEM((1, N), jnp.float32),
            pltpu.SemaphoreType.DMA,
        ],
        compiler_params=pltpu.CompilerParams(
            dimension_semantics=("arbitrary", "arbitrary"),
        ),
    )(inputs, emb, W1, b1r, W2, b2r)
    return out.reshape(1, PADV)[:, :V]


# speedup vs baseline: 1.7867x; 1.3826x over previous
import jax
import jax.numpy as jnp
from jax import lax
from jax.experimental import pallas as pl
from jax.experimental.pallas import tpu as pltpu

V = 100000
D = 128
C = 20
N = 128
VB = 8192
NBLK = (V + VB - 1) // VB
PADV = NBLK * VB


def _fused_kernel(idx_ref, emb_ref, w1_ref, b1_ref, w2_ref, b2_ref,
                  out_ref, g_ref, h_ref, sem):
    i = pl.program_id(0)

    @pl.when(i == 0)
    def _gather_and_hidden():
        for p in range(C):
            pltpu.make_async_copy(
                emb_ref.at[pl.ds(idx_ref[p], 1), :],
                g_ref.at[pl.ds(p, 1), :],
                sem,
            ).start()
        for p in range(C):
            pltpu.make_async_copy(
                emb_ref.at[pl.ds(idx_ref[p], 1), :],
                g_ref.at[pl.ds(p, 1), :],
                sem,
            ).wait()
        acc = b1_ref[...].astype(jnp.float32)
        for p in range(C):
            acc = acc + lax.dot_general(
                g_ref[pl.ds(p, 1), :],
                w1_ref[:, pl.ds(p * D, D)],
                (((1,), (1,)), ((), ())),
                preferred_element_type=jnp.float32,
            )
        h_ref[...] = jnp.maximum(acc, 0.0)

    lb = lax.dot_general(
        h_ref[...],
        w2_ref[...],
        (((1,), (1,)), ((), ())),
        preferred_element_type=jnp.float32,
    ) + b2_ref[...]

    @pl.when(i < NBLK - 1)
    def _store():
        out_ref[0:1, pl.ds(i * VB, VB)] = lb

    @pl.when(i == NBLK - 1)
    def _store_masked_and_normalize():
        cols = (NBLK - 1) * VB + lax.broadcasted_iota(jnp.int32, (1, VB), 1)
        out_ref[0:1, pl.ds((NBLK - 1) * VB, VB)] = jnp.where(cols < V, lb, -1e30)
        scr = out_ref[...]
        m = jnp.max(scr, axis=1, keepdims=True)
        s = jnp.sum(jnp.exp(scr - m), axis=1, keepdims=True)
        out_ref[...] = scr - (m + jnp.log(s))


def kernel(inputs, emb, W1, b1, W2, b2):
    b1r = b1.reshape(1, N)
    b2r = b2.reshape(1, V)
    out = pl.pallas_call(
        _fused_kernel,
        grid=(NBLK,),
        in_specs=[
            pl.BlockSpec(memory_space=pltpu.MemorySpace.SMEM),
            pl.BlockSpec(memory_space=pltpu.MemorySpace.HBM),
            pl.BlockSpec((N, C * D), lambda i: (0, 0)),
            pl.BlockSpec((1, N), lambda i: (0, 0)),
            pl.BlockSpec((VB, D), lambda i: (i, 0)),
            pl.BlockSpec((1, VB), lambda i: (0, i)),
        ],
        out_specs=pl.BlockSpec((1, PADV), lambda i: (0, 0)),
        out_shape=jax.ShapeDtypeStruct((1, PADV), jnp.float32),
        scratch_shapes=[
            pltpu.VMEM((C, D), jnp.float32),
            pltpu.VMEM((1, N), jnp.float32),
            pltpu.SemaphoreType.DMA,
        ],
        compiler_params=pltpu.CompilerParams(
            dimension_semantics=("arbitrary",),
            vmem_limit_bytes=100 * 1024 * 1024,
        ),
    )(inputs, emb, W1, b1r, W2, b2r)
    return out[:, :V]


# speedup vs baseline: 1.8443x; 1.0322x over previous
import jax
import jax.numpy as jnp
from jax import lax
from jax.experimental import pallas as pl
from jax.experimental.pallas import tpu as pltpu

V = 100000
D = 128
C = 20
N = 128
VB = 16384
NBLK = (V + VB - 1) // VB
PADV = NBLK * VB


def _fused_kernel(idx_ref, emb_ref, w1_ref, b1_ref, w2_ref, b2_ref,
                  out_ref, g_ref, h_ref, sem):
    i = pl.program_id(0)

    @pl.when(i == 0)
    def _gather_and_hidden():
        for p in range(C):
            pltpu.make_async_copy(
                emb_ref.at[pl.ds(idx_ref[p], 1), :],
                g_ref.at[pl.ds(p, 1), :],
                sem,
            ).start()
        for p in range(C):
            pltpu.make_async_copy(
                emb_ref.at[pl.ds(idx_ref[p], 1), :],
                g_ref.at[pl.ds(p, 1), :],
                sem,
            ).wait()
        acc = b1_ref[...].astype(jnp.float32)
        for p in range(C):
            acc = acc + lax.dot_general(
                g_ref[pl.ds(p, 1), :],
                w1_ref[:, pl.ds(p * D, D)],
                (((1,), (1,)), ((), ())),
                preferred_element_type=jnp.float32,
            )
        h_ref[...] = jnp.maximum(acc, 0.0)

    lb = lax.dot_general(
        h_ref[...],
        w2_ref[...],
        (((1,), (1,)), ((), ())),
        preferred_element_type=jnp.float32,
    ) + b2_ref[...]

    @pl.when(i < NBLK - 1)
    def _store():
        out_ref[0:1, pl.ds(i * VB, VB)] = lb

    @pl.when(i == NBLK - 1)
    def _store_masked_and_normalize():
        cols = (NBLK - 1) * VB + lax.broadcasted_iota(jnp.int32, (1, VB), 1)
        out_ref[0:1, pl.ds((NBLK - 1) * VB, VB)] = jnp.where(cols < V, lb, -1e30)
        scr = out_ref[...]
        m = jnp.max(scr, axis=1, keepdims=True)
        s = jnp.sum(jnp.exp(scr - m), axis=1, keepdims=True)
        out_ref[...] = scr - (m + jnp.log(s))


def kernel(inputs, emb, W1, b1, W2, b2):
    b1r = b1.reshape(1, N)
    b2r = b2.reshape(1, V)
    out = pl.pallas_call(
        _fused_kernel,
        grid=(NBLK,),
        in_specs=[
            pl.BlockSpec(memory_space=pltpu.MemorySpace.SMEM),
            pl.BlockSpec(memory_space=pltpu.MemorySpace.HBM),
            pl.BlockSpec((N, C * D), lambda i: (0, 0)),
            pl.BlockSpec((1, N), lambda i: (0, 0)),
            pl.BlockSpec((VB, D), lambda i: (i, 0)),
            pl.BlockSpec((1, VB), lambda i: (0, i)),
        ],
        out_specs=pl.BlockSpec((1, PADV), lambda i: (0, 0)),
        out_shape=jax.ShapeDtypeStruct((1, PADV), jnp.float32),
        scratch_shapes=[
            pltpu.VMEM((C, D), jnp.float32),
            pltpu.VMEM((1, N), jnp.float32),
            pltpu.SemaphoreType.DMA,
        ],
        compiler_params=pltpu.CompilerParams(
            dimension_semantics=("arbitrary",),
            vmem_limit_bytes=100 * 1024 * 1024,
        ),
    )(inputs, emb, W1, b1r, W2, b2r)
    return out[:, :V]
